# baseline (speedup 1.0000x reference)
import functools

import jax
import jax.numpy as jnp
from jax import lax
from jax.experimental import pallas as pl
from jax.experimental.pallas import tpu as pltpu
from jax.experimental.pallas import tpu_sc as plsc

N = 8192
K = 8192
D = 256
TI = 256
CH = 1024
KB = 2048
HW = 1024
B = 8
LAG = 4


def _enc_copy(slab_ref, enc_ref, sem_ref, j):
    s = j % 2
    b = j // LAG
    kb = j % LAG
    return pltpu.make_async_copy(
        slab_ref.at[s],
        enc_ref.at[b, pl.ds(kb * KB, KB), :],
        sem_ref.at[s],
    )


def fused_body(x_ref, cb_ref, x2_ref, y2_ref,
               idx_ref, enc_ref, cnt_ref, iscr_ref, cacc_ref, slab_ref, sem_ref):
    i = pl.program_id(0)
    nhot = N // TI

    @pl.when(i < nhot)
    def _argmin():
        _argmin_step(i, x_ref, cb_ref, x2_ref, y2_ref, idx_ref, iscr_ref)

    @pl.when((i >= LAG + 2) & (i < nhot + LAG + 2))
    def _drain():
        _enc_copy(slab_ref, enc_ref, sem_ref, i - LAG - 2).wait()

    @pl.when((i >= LAG) & (i < nhot + LAG))
    def _onehot():
        j = i - LAG
        b = j // LAG
        kb = j % LAG
        s = j % 2
        kr = kb * KB + lax.broadcasted_iota(jnp.int32, (KB, 1), 0)
        rs = jnp.zeros((KB, 1), jnp.float32)
        for q in range(LAG):
            ids_q = iscr_ref[pl.ds(LAG * b + q, 1)][0]
            oh_q = (ids_q == kr).astype(jnp.float32)
            slab_ref[pl.ds(s, 1), :, q * TI:(q + 1) * TI] = oh_q.reshape(1, KB, TI)
            rs = rs + lax.dot_general(oh_q, jnp.ones((TI, 1), jnp.float32),
                                      (((1,), (0,)), ((), ())),
                                      preferred_element_type=jnp.float32)
        _enc_copy(slab_ref, enc_ref, sem_ref, j).start()
        sl = pl.ds(pl.multiple_of(kb * KB, KB), KB)

        @pl.when(b == 0)
        def _():
            cacc_ref[sl, :] = rs

        @pl.when(b != 0)
        def _():
            cacc_ref[sl, :] = cacc_ref[sl, :] + rs

        @pl.when(j == nhot - 1)
        def _():
            cnt_ref[...] = cacc_ref[...]


def _argmin_step(i, x_ref, cb_ref, x2_ref, y2_ref, idx_ref, iscr_ref):
    x = x_ref[...]
    x2 = x2_ref[0]
    nch = K // CH

    def dot_chunk(jc):
        y = cb_ref[pl.ds(pl.multiple_of(jc * CH, CH), CH), :]
        return lax.dot_general(y, x, (((1,), (1,)), ((), ())),
                               preferred_element_type=jnp.float32)

    def chunk(jc, carry):
        mv, mi, xy = carry
        xy_next = dot_chunk(jnp.minimum(jc + 1, nch - 1))
        y2 = y2_ref[pl.ds(pl.multiple_of(jc * CH, CH), CH), :]
        d2 = (x2 - 2.0 * xy) + y2
        s = jnp.sqrt(d2)
        cmin = jnp.min(s, axis=0, keepdims=True)
        ii = lax.broadcasted_iota(jnp.int32, (CH, TI), 0) + jc * CH
        cidx = jnp.min(jnp.where(s == cmin, ii, jnp.int32(2**30)),
                       axis=0, keepdims=True)
        take = cmin < mv
        return (jnp.where(take, cmin, mv), jnp.where(take, cidx, mi), xy_next)

    mv0 = jnp.full((1, TI), jnp.inf, jnp.float32)
    mi0 = jnp.zeros((1, TI), jnp.int32)
    _, mi, _ = lax.fori_loop(0, nch, chunk, (mv0, mi0, dot_chunk(0)))
    idx_ref[0] = mi
    iscr_ref[pl.ds(i, 1)] = mi.reshape(1, 1, TI)


def _fused_call(flat, codebook, x2, y2):
    last = N // TI - 1

    def clamp2(i):
        return (jnp.minimum(i, last), 0)

    def clamp3(i):
        return (jnp.minimum(i, last), 0, 0)

    return pl.pallas_call(
        fused_body,
        grid=(N // TI + LAG + 2,),
        in_specs=[
            pl.BlockSpec((TI, D), clamp2),
            pl.BlockSpec((K, D), lambda i: (0, 0)),
            pl.BlockSpec((1, 1, TI), clamp3),
            pl.BlockSpec((K, 1), lambda i: (0, 0)),
        ],
        out_specs=[
            pl.BlockSpec((1, 1, TI), clamp3),
            pl.BlockSpec(memory_space=pl.ANY),
            pl.BlockSpec((K, 1), lambda i: (0, 0)),
        ],
        out_shape=[
            jax.ShapeDtypeStruct((N // TI, 1, TI), jnp.int32),
            jax.ShapeDtypeStruct((B, K, HW), jnp.float32),
            jax.ShapeDtypeStruct((K, 1), jnp.float32),
        ],
        scratch_shapes=[
            pltpu.VMEM((N // TI, 1, TI), jnp.int32),
            pltpu.VMEM((K, 1), jnp.float32),
            pltpu.VMEM((2, KB, HW), jnp.float32),
            pltpu.SemaphoreType.DMA((2,)),
        ],
    )(flat, codebook, x2, y2)


def qst_body(x_ref, q_ref, cnt_ref, o_ref, loss_ref, perp_ref, acc_ref):
    b = pl.program_id(0)
    x = x_ref[0]
    q = q_ref[0]
    qst = x + (q - x)
    o_ref[0] = qst
    diff = qst - x
    part = jnp.sum(diff * diff)

    @pl.when(b == 0)
    def _():
        acc_ref[0, 0] = part

    @pl.when(b != 0)
    def _():
        acc_ref[0, 0] = acc_ref[0, 0] + part

    @pl.when(b == pl.num_programs(0) - 1)
    def _():
        m = acc_ref[0, 0] / jnp.float32(N * D)
        loss_ref[0, 0] = m + 0.25 * m
        p = cnt_ref[...] * jnp.float32(1.0 / N)
        t = p * jnp.log(p + 1e-10)
        perp_ref[0, 0] = jnp.exp(-jnp.sum(t))


def _qst_call(flat3, q3, cnt):
    return pl.pallas_call(
        qst_body,
        grid=(B,),
        in_specs=[
            pl.BlockSpec((1, HW, D), lambda b: (b, 0, 0)),
            pl.BlockSpec((1, HW, D), lambda b: (b, 0, 0)),
            pl.BlockSpec((K // 128, 128), lambda b: (0, 0)),
        ],
        out_specs=[
            pl.BlockSpec((1, HW, D), lambda b: (b, 0, 0)),
            pl.BlockSpec(memory_space=pltpu.SMEM),
            pl.BlockSpec(memory_space=pltpu.SMEM),
        ],
        out_shape=[
            jax.ShapeDtypeStruct((B, HW, D), jnp.float32),
            jax.ShapeDtypeStruct((1, 1), jnp.float32),
            jax.ShapeDtypeStruct((1, 1), jnp.float32),
        ],
        scratch_shapes=[pltpu.SMEM((1, 1), jnp.float32)],
    )(flat3, q3, cnt)


def _sc_gather(codebook, idx):
    info = plsc.get_sparse_core_info()
    nc, ns = info.num_cores, info.num_subcores
    nw = nc * ns
    bpw = N // nw
    mesh = plsc.VectorSubcoreMesh(core_axis_name="c", subcore_axis_name="s")

    @functools.partial(
        pl.kernel,
        mesh=mesh,
        out_type=jax.ShapeDtypeStruct((N, D), jnp.float32),
        scratch_types=[
            pltpu.VMEM((bpw,), jnp.int32),
            pltpu.VMEM((bpw, D), jnp.float32),
            pltpu.SemaphoreType.DMA,
        ],
    )
    def gather_k(table_hbm, idx_hbm, out_hbm, idx_v, rows_v, sem):
        wid = lax.axis_index("s") * nc + lax.axis_index("c")
        base = wid * bpw
        pltpu.sync_copy(idx_hbm.at[pl.ds(base, bpw)], idx_v)
        pltpu.async_copy(table_hbm.at[idx_v], rows_v, sem).wait()
        pltpu.sync_copy(rows_v, out_hbm.at[pl.ds(base, bpw)])

    return gather_k(codebook, idx)


def kernel(inputs, codebook):
    flat = jnp.transpose(inputs, (0, 2, 3, 1)).reshape(-1, D)
    x2 = jnp.sum(flat * flat, axis=1, keepdims=True).reshape(N // TI, 1, TI)
    y2 = jnp.sum(codebook * codebook, axis=1).reshape(K, 1)

    idx, enc, cnt = _fused_call(flat, codebook, x2, y2)

    qflat = _sc_gather(codebook, idx.reshape(N))
    qst3, loss, perp = _qst_call(flat.reshape(B, HW, D), qflat.reshape(B, HW, D),
                                 cnt.reshape(K // 128, 128))

    quantized_st = jnp.transpose(qst3.reshape(B, 32, 32, D), (0, 3, 1, 2))
    encodings_out = enc.reshape(B, K, 32, 32)
    return (quantized_st, loss.reshape(()), perp.reshape(()), encodings_out)


# speedup vs baseline: 1.1520x; 1.1520x over previous
import functools

import jax
import jax.numpy as jnp
from jax import lax
from jax.experimental import pallas as pl
from jax.experimental.pallas import tpu as pltpu
from jax.experimental.pallas import tpu_sc as plsc

N = 8192
K = 8192
D = 256
TI = 256
CH = 2048
KB = 2048
HW = 1024
B = 8
LAG = 4


def _enc_copy(slab_ref, enc_ref, sem_ref, j):
    s = j % 2
    b = j // LAG
    kb = j % LAG
    return pltpu.make_async_copy(
        slab_ref.at[s],
        enc_ref.at[b, pl.ds(kb * KB, KB), :],
        sem_ref.at[s],
    )


def fused_body(x_ref, cb_ref, x2_ref, y2_ref,
               idx_ref, enc_ref, cnt_ref, iscr_ref, cacc_ref, slab_ref, sem_ref):
    i = pl.program_id(0)
    nhot = N // TI

    @pl.when(i < nhot)
    def _argmin():
        _argmin_step(i, x_ref, cb_ref, x2_ref, y2_ref, idx_ref, iscr_ref)

    @pl.when((i >= LAG + 2) & (i < nhot + LAG + 2))
    def _drain():
        _enc_copy(slab_ref, enc_ref, sem_ref, i - LAG - 2).wait()

    @pl.when((i >= LAG) & (i < nhot + LAG))
    def _onehot():
        j = i - LAG
        b = j // LAG
        kb = j % LAG
        s = j % 2
        kr = kb * KB + lax.broadcasted_iota(jnp.int32, (KB, 1), 0)
        rs = jnp.zeros((KB, 1), jnp.float32)
        for q in range(LAG):
            ids_q = iscr_ref[pl.ds(LAG * b + q, 1)][0]
            oh_q = (ids_q == kr).astype(jnp.float32)
            slab_ref[pl.ds(s, 1), :, q * TI:(q + 1) * TI] = oh_q.reshape(1, KB, TI)
            rs = rs + lax.dot_general(oh_q, jnp.ones((TI, 1), jnp.float32),
                                      (((1,), (0,)), ((), ())),
                                      preferred_element_type=jnp.float32)
        _enc_copy(slab_ref, enc_ref, sem_ref, j).start()
        sl = pl.ds(pl.multiple_of(kb * KB, KB), KB)

        @pl.when(b == 0)
        def _():
            cacc_ref[sl, :] = rs

        @pl.when(b != 0)
        def _():
            cacc_ref[sl, :] = cacc_ref[sl, :] + rs

        @pl.when(j == nhot - 1)
        def _():
            cnt_ref[...] = cacc_ref[...]


def _argmin_step(i, x_ref, cb_ref, x2_ref, y2_ref, idx_ref, iscr_ref):
    x = x_ref[...]
    x2 = x2_ref[0]

    def chunk(jc, carry):
        mv, mi = carry
        y = cb_ref[pl.ds(pl.multiple_of(jc * CH, CH), CH), :]
        y2 = y2_ref[pl.ds(pl.multiple_of(jc * CH, CH), CH), :]
        xy = lax.dot_general(y, x, (((1,), (1,)), ((), ())),
                             preferred_element_type=jnp.float32)
        d2 = (x2 - 2.0 * xy) + y2
        s = jnp.sqrt(d2)
        cmin = jnp.min(s, axis=0, keepdims=True)
        ii = lax.broadcasted_iota(jnp.int32, (CH, TI), 0) + jc * CH
        cidx = jnp.min(jnp.where(s == cmin, ii, jnp.int32(2**30)),
                       axis=0, keepdims=True)
        take = cmin < mv
        return jnp.where(take, cmin, mv), jnp.where(take, cidx, mi)

    mv0 = jnp.full((1, TI), jnp.inf, jnp.float32)
    mi0 = jnp.zeros((1, TI), jnp.int32)
    _, mi = lax.fori_loop(0, K // CH, chunk, (mv0, mi0))
    idx_ref[0] = mi
    iscr_ref[pl.ds(i, 1)] = mi.reshape(1, 1, TI)


def _fused_call(flat, codebook, x2, y2):
    last = N // TI - 1

    def clamp2(i):
        return (jnp.minimum(i, last), 0)

    def clamp3(i):
        return (jnp.minimum(i, last), 0, 0)

    return pl.pallas_call(
        fused_body,
        grid=(N // TI + LAG + 2,),
        in_specs=[
            pl.BlockSpec((TI, D), clamp2),
            pl.BlockSpec((K, D), lambda i: (0, 0)),
            pl.BlockSpec((1, 1, TI), clamp3),
            pl.BlockSpec((K, 1), lambda i: (0, 0)),
        ],
        out_specs=[
            pl.BlockSpec((1, 1, TI), clamp3),
            pl.BlockSpec(memory_space=pl.ANY),
            pl.BlockSpec((K, 1), lambda i: (0, 0)),
        ],
        out_shape=[
            jax.ShapeDtypeStruct((N // TI, 1, TI), jnp.int32),
            jax.ShapeDtypeStruct((B, K, HW), jnp.float32),
            jax.ShapeDtypeStruct((K, 1), jnp.float32),
        ],
        scratch_shapes=[
            pltpu.VMEM((N // TI, 1, TI), jnp.int32),
            pltpu.VMEM((K, 1), jnp.float32),
            pltpu.VMEM((2, KB, HW), jnp.float32),
            pltpu.SemaphoreType.DMA((2,)),
        ],
    )(flat, codebook, x2, y2)


def qst_body(x_ref, q_ref, cnt_ref, o_ref, loss_ref, perp_ref, acc_ref):
    b = pl.program_id(0)
    x = x_ref[0]
    q = q_ref[0]
    qst = x + (q - x)
    o_ref[0] = qst
    diff = qst - x
    part = jnp.sum(diff * diff)

    @pl.when(b == 0)
    def _():
        acc_ref[0, 0] = part

    @pl.when(b != 0)
    def _():
        acc_ref[0, 0] = acc_ref[0, 0] + part

    @pl.when(b == pl.num_programs(0) - 1)
    def _():
        m = acc_ref[0, 0] / jnp.float32(N * D)
        loss_ref[0, 0] = m + 0.25 * m
        p = cnt_ref[...] * jnp.float32(1.0 / N)
        t = p * jnp.log(p + 1e-10)
        perp_ref[0, 0] = jnp.exp(-jnp.sum(t))


def _qst_call(flat3, q3, cnt):
    return pl.pallas_call(
        qst_body,
        grid=(B,),
        in_specs=[
            pl.BlockSpec((1, HW, D), lambda b: (b, 0, 0)),
            pl.BlockSpec((1, HW, D), lambda b: (b, 0, 0)),
            pl.BlockSpec((K // 128, 128), lambda b: (0, 0)),
        ],
        out_specs=[
            pl.BlockSpec((1, HW, D), lambda b: (b, 0, 0)),
            pl.BlockSpec(memory_space=pltpu.SMEM),
            pl.BlockSpec(memory_space=pltpu.SMEM),
        ],
        out_shape=[
            jax.ShapeDtypeStruct((B, HW, D), jnp.float32),
            jax.ShapeDtypeStruct((1, 1), jnp.float32),
            jax.ShapeDtypeStruct((1, 1), jnp.float32),
        ],
        scratch_shapes=[pltpu.SMEM((1, 1), jnp.float32)],
    )(flat3, q3, cnt)


def _sc_gather(codebook, idx):
    info = plsc.get_sparse_core_info()
    nc, ns = info.num_cores, info.num_subcores
    nw = nc * ns
    bpw = N // nw
    mesh = plsc.VectorSubcoreMesh(core_axis_name="c", subcore_axis_name="s")

    @functools.partial(
        pl.kernel,
        mesh=mesh,
        out_type=jax.ShapeDtypeStruct((N, D), jnp.float32),
        scratch_types=[
            pltpu.VMEM((bpw,), jnp.int32),
            pltpu.VMEM((bpw, D), jnp.float32),
            pltpu.SemaphoreType.DMA,
        ],
    )
    def gather_k(table_hbm, idx_hbm, out_hbm, idx_v, rows_v, sem):
        wid = lax.axis_index("s") * nc + lax.axis_index("c")
        base = wid * bpw
        pltpu.sync_copy(idx_hbm.at[pl.ds(base, bpw)], idx_v)
        pltpu.async_copy(table_hbm.at[idx_v], rows_v, sem).wait()
        pltpu.sync_copy(rows_v, out_hbm.at[pl.ds(base, bpw)])

    return gather_k(codebook, idx)


def kernel(inputs, codebook):
    flat = jnp.transpose(inputs, (0, 2, 3, 1)).reshape(-1, D)
    x2 = jnp.sum(flat * flat, axis=1, keepdims=True).reshape(N // TI, 1, TI)
    y2 = jnp.sum(codebook * codebook, axis=1).reshape(K, 1)

    idx, enc, cnt = _fused_call(flat, codebook, x2, y2)

    qflat = _sc_gather(codebook, idx.reshape(N))
    qst3, loss, perp = _qst_call(flat.reshape(B, HW, D), qflat.reshape(B, HW, D),
                                 cnt.reshape(K // 128, 128))

    quantized_st = jnp.transpose(qst3.reshape(B, 32, 32, D), (0, 3, 1, 2))
    encodings_out = enc.reshape(B, K, 32, 32)
    return (quantized_st, loss.reshape(()), perp.reshape(()), encodings_out)


# speedup vs baseline: 1.1793x; 1.0236x over previous
import functools

import jax
import jax.numpy as jnp
from jax import lax
from jax.experimental import pallas as pl
from jax.experimental.pallas import tpu as pltpu
from jax.experimental.pallas import tpu_sc as plsc

N = 8192
K = 8192
D = 256
TI = 256
CH = 4096
KB = 2048
HW = 1024
B = 8
LAG = 4


def _enc_copy(slab_ref, enc_ref, sem_ref, j):
    s = j % 2
    b = j // LAG
    kb = j % LAG
    return pltpu.make_async_copy(
        slab_ref.at[s],
        enc_ref.at[b, pl.ds(kb * KB, KB), :],
        sem_ref.at[s],
    )


def fused_body(x_ref, cb_ref, x2_ref, y2_ref,
               idx_ref, enc_ref, cnt_ref, iscr_ref, cacc_ref, slab_ref, sem_ref):
    i = pl.program_id(0)
    nhot = N // TI

    @pl.when(i < nhot)
    def _argmin():
        _argmin_step(i, x_ref, cb_ref, x2_ref, y2_ref, idx_ref, iscr_ref)

    @pl.when((i >= LAG + 2) & (i < nhot + LAG + 2))
    def _drain():
        _enc_copy(slab_ref, enc_ref, sem_ref, i - LAG - 2).wait()

    @pl.when((i >= LAG) & (i < nhot + LAG))
    def _onehot():
        j = i - LAG
        b = j // LAG
        kb = j % LAG
        s = j % 2
        kr = kb * KB + lax.broadcasted_iota(jnp.int32, (KB, 1), 0)
        rs = jnp.zeros((KB, 1), jnp.float32)
        for q in range(LAG):
            ids_q = iscr_ref[pl.ds(LAG * b + q, 1)][0]
            oh_q = (ids_q == kr).astype(jnp.float32)
            slab_ref[pl.ds(s, 1), :, q * TI:(q + 1) * TI] = oh_q.reshape(1, KB, TI)
            rs = rs + lax.dot_general(oh_q, jnp.ones((TI, 1), jnp.float32),
                                      (((1,), (0,)), ((), ())),
                                      preferred_element_type=jnp.float32)
        _enc_copy(slab_ref, enc_ref, sem_ref, j).start()
        sl = pl.ds(pl.multiple_of(kb * KB, KB), KB)

        @pl.when(b == 0)
        def _():
            cacc_ref[sl, :] = rs

        @pl.when(b != 0)
        def _():
            cacc_ref[sl, :] = cacc_ref[sl, :] + rs

        @pl.when(j == nhot - 1)
        def _():
            cnt_ref[...] = cacc_ref[...]


def _argmin_step(i, x_ref, cb_ref, x2_ref, y2_ref, idx_ref, iscr_ref):
    x = x_ref[...]
    x2 = x2_ref[0]

    def chunk(jc, carry):
        mv, mi = carry
        y = cb_ref[pl.ds(pl.multiple_of(jc * CH, CH), CH), :]
        y2 = y2_ref[pl.ds(pl.multiple_of(jc * CH, CH), CH), :]
        xy = lax.dot_general(y, x, (((1,), (1,)), ((), ())),
                             preferred_element_type=jnp.float32)
        d2 = (x2 - 2.0 * xy) + y2
        s = jnp.sqrt(d2)
        cmin = jnp.min(s, axis=0, keepdims=True)
        ii = lax.broadcasted_iota(jnp.int32, (CH, TI), 0) + jc * CH
        cidx = jnp.min(jnp.where(s == cmin, ii, jnp.int32(2**30)),
                       axis=0, keepdims=True)
        take = cmin < mv
        return jnp.where(take, cmin, mv), jnp.where(take, cidx, mi)

    mv0 = jnp.full((1, TI), jnp.inf, jnp.float32)
    mi0 = jnp.zeros((1, TI), jnp.int32)
    _, mi = lax.fori_loop(0, K // CH, chunk, (mv0, mi0))
    idx_ref[0] = mi
    iscr_ref[pl.ds(i, 1)] = mi.reshape(1, 1, TI)


def _fused_call(flat, codebook, x2, y2):
    last = N // TI - 1

    def clamp2(i):
        return (jnp.minimum(i, last), 0)

    def clamp3(i):
        return (jnp.minimum(i, last), 0, 0)

    return pl.pallas_call(
        fused_body,
        grid=(N // TI + LAG + 2,),
        in_specs=[
            pl.BlockSpec((TI, D), clamp2),
            pl.BlockSpec((K, D), lambda i: (0, 0)),
            pl.BlockSpec((1, 1, TI), clamp3),
            pl.BlockSpec((K, 1), lambda i: (0, 0)),
        ],
        out_specs=[
            pl.BlockSpec((1, 1, TI), clamp3),
            pl.BlockSpec(memory_space=pl.ANY),
            pl.BlockSpec((K, 1), lambda i: (0, 0)),
        ],
        out_shape=[
            jax.ShapeDtypeStruct((N // TI, 1, TI), jnp.int32),
            jax.ShapeDtypeStruct((B, K, HW), jnp.float32),
            jax.ShapeDtypeStruct((K, 1), jnp.float32),
        ],
        scratch_shapes=[
            pltpu.VMEM((N // TI, 1, TI), jnp.int32),
            pltpu.VMEM((K, 1), jnp.float32),
            pltpu.VMEM((2, KB, HW), jnp.float32),
            pltpu.SemaphoreType.DMA((2,)),
        ],
    )(flat, codebook, x2, y2)


def qst_body(x_ref, q_ref, cnt_ref, o_ref, loss_ref, perp_ref, acc_ref):
    b = pl.program_id(0)
    x = x_ref[0]
    q = q_ref[0]
    qst = x + (q - x)
    o_ref[0] = qst
    diff = qst - x
    part = jnp.sum(diff * diff)

    @pl.when(b == 0)
    def _():
        acc_ref[0, 0] = part

    @pl.when(b != 0)
    def _():
        acc_ref[0, 0] = acc_ref[0, 0] + part

    @pl.when(b == pl.num_programs(0) - 1)
    def _():
        m = acc_ref[0, 0] / jnp.float32(N * D)
        loss_ref[0, 0] = m + 0.25 * m
        p = cnt_ref[...] * jnp.float32(1.0 / N)
        t = p * jnp.log(p + 1e-10)
        perp_ref[0, 0] = jnp.exp(-jnp.sum(t))


def _qst_call(flat3, q3, cnt):
    return pl.pallas_call(
        qst_body,
        grid=(B,),
        in_specs=[
            pl.BlockSpec((1, HW, D), lambda b: (b, 0, 0)),
            pl.BlockSpec((1, HW, D), lambda b: (b, 0, 0)),
            pl.BlockSpec((K // 128, 128), lambda b: (0, 0)),
        ],
        out_specs=[
            pl.BlockSpec((1, HW, D), lambda b: (b, 0, 0)),
            pl.BlockSpec(memory_space=pltpu.SMEM),
            pl.BlockSpec(memory_space=pltpu.SMEM),
        ],
        out_shape=[
            jax.ShapeDtypeStruct((B, HW, D), jnp.float32),
            jax.ShapeDtypeStruct((1, 1), jnp.float32),
            jax.ShapeDtypeStruct((1, 1), jnp.float32),
        ],
        scratch_shapes=[pltpu.SMEM((1, 1), jnp.float32)],
    )(flat3, q3, cnt)


def _sc_gather(codebook, idx):
    info = plsc.get_sparse_core_info()
    nc, ns = info.num_cores, info.num_subcores
    nw = nc * ns
    bpw = N // nw
    mesh = plsc.VectorSubcoreMesh(core_axis_name="c", subcore_axis_name="s")

    @functools.partial(
        pl.kernel,
        mesh=mesh,
        out_type=jax.ShapeDtypeStruct((N, D), jnp.float32),
        scratch_types=[
            pltpu.VMEM((bpw,), jnp.int32),
            pltpu.VMEM((bpw, D), jnp.float32),
            pltpu.SemaphoreType.DMA,
        ],
    )
    def gather_k(table_hbm, idx_hbm, out_hbm, idx_v, rows_v, sem):
        wid = lax.axis_index("s") * nc + lax.axis_index("c")
        base = wid * bpw
        pltpu.sync_copy(idx_hbm.at[pl.ds(base, bpw)], idx_v)
        pltpu.async_copy(table_hbm.at[idx_v], rows_v, sem).wait()
        pltpu.sync_copy(rows_v, out_hbm.at[pl.ds(base, bpw)])

    return gather_k(codebook, idx)


def kernel(inputs, codebook):
    flat = jnp.transpose(inputs, (0, 2, 3, 1)).reshape(-1, D)
    x2 = jnp.sum(flat * flat, axis=1, keepdims=True).reshape(N // TI, 1, TI)
    y2 = jnp.sum(codebook * codebook, axis=1).reshape(K, 1)

    idx, enc, cnt = _fused_call(flat, codebook, x2, y2)

    qflat = _sc_gather(codebook, idx.reshape(N))
    qst3, loss, perp = _qst_call(flat.reshape(B, HW, D), qflat.reshape(B, HW, D),
                                 cnt.reshape(K // 128, 128))

    quantized_st = jnp.transpose(qst3.reshape(B, 32, 32, D), (0, 3, 1, 2))
    encodings_out = enc.reshape(B, K, 32, 32)
    return (quantized_st, loss.reshape(()), perp.reshape(()), encodings_out)


# speedup vs baseline: 1.2169x; 1.0319x over previous
import functools

import jax
import jax.numpy as jnp
from jax import lax
from jax.experimental import pallas as pl
from jax.experimental.pallas import tpu as pltpu
from jax.experimental.pallas import tpu_sc as plsc

N = 8192
K = 8192
D = 256
TI = 256
CH = 4096
KB = 2048
HW = 1024
B = 8
LAG = 4


def _enc_copy(slab_ref, enc_ref, sem_ref, j):
    s = j % 2
    b = j // LAG
    kb = j % LAG
    return pltpu.make_async_copy(
        slab_ref.at[s],
        enc_ref.at[b, pl.ds(kb * KB, KB), :],
        sem_ref.at[s],
    )


def fused_body(x_ref, cb_ref, x2_ref, y2_ref,
               idx_ref, enc_ref, cnt_ref, iscr_ref, cacc_ref, slab_ref, sem_ref):
    i = pl.program_id(0)
    nhot = N // TI

    @pl.when(i < nhot)
    def _argmin():
        _argmin_step(i, x_ref, cb_ref, x2_ref, y2_ref, idx_ref, iscr_ref)

    @pl.when((i >= LAG + 2) & (i < nhot + LAG + 2))
    def _drain():
        _enc_copy(slab_ref, enc_ref, sem_ref, i - LAG - 2).wait()

    @pl.when((i >= LAG) & (i < nhot + LAG))
    def _onehot():
        j = i - LAG
        b = j // LAG
        kb = j % LAG
        s = j % 2
        kr = kb * KB + lax.broadcasted_iota(jnp.int32, (KB, 1), 0)
        rs = jnp.zeros((KB, 1), jnp.float32)
        for q in range(LAG):
            ids_q = iscr_ref[pl.ds(LAG * b + q, 1)][0]
            oh_q = (ids_q == kr).astype(jnp.float32)
            slab_ref[pl.ds(s, 1), :, q * TI:(q + 1) * TI] = oh_q.reshape(1, KB, TI)
            rs = rs + lax.dot_general(oh_q, jnp.ones((TI, 1), jnp.float32),
                                      (((1,), (0,)), ((), ())),
                                      preferred_element_type=jnp.float32)
        _enc_copy(slab_ref, enc_ref, sem_ref, j).start()
        sl = pl.ds(pl.multiple_of(kb * KB, KB), KB)

        @pl.when(b == 0)
        def _():
            cacc_ref[sl, :] = rs

        @pl.when(b != 0)
        def _():
            cacc_ref[sl, :] = cacc_ref[sl, :] + rs

        @pl.when(j == nhot - 1)
        def _():
            cnt_ref[...] = cacc_ref[...]


def _argmin_step(i, x_ref, cb_ref, x2_ref, y2_ref, idx_ref, iscr_ref):
    x = x_ref[...]
    x2 = x2_ref[0]

    xys = [lax.dot_general(cb_ref[pl.ds(jc * CH, CH), :], x,
                           (((1,), (1,)), ((), ())),
                           preferred_element_type=jnp.float32)
           for jc in range(K // CH)]
    mv = jnp.full((1, TI), jnp.inf, jnp.float32)
    mi = jnp.zeros((1, TI), jnp.int32)
    for jc in range(K // CH):
        y2 = y2_ref[pl.ds(jc * CH, CH), :]
        d2 = (x2 - 2.0 * xys[jc]) + y2
        s = jnp.sqrt(d2)
        cmin = jnp.min(s, axis=0, keepdims=True)
        ii = lax.broadcasted_iota(jnp.int32, (CH, TI), 0) + jc * CH
        cidx = jnp.min(jnp.where(s == cmin, ii, jnp.int32(2**30)),
                       axis=0, keepdims=True)
        take = cmin < mv
        mv = jnp.where(take, cmin, mv)
        mi = jnp.where(take, cidx, mi)
    idx_ref[0] = mi
    iscr_ref[pl.ds(i, 1)] = mi.reshape(1, 1, TI)


def _fused_call(flat, codebook, x2, y2):
    last = N // TI - 1

    def clamp2(i):
        return (jnp.minimum(i, last), 0)

    def clamp3(i):
        return (jnp.minimum(i, last), 0, 0)

    return pl.pallas_call(
        fused_body,
        grid=(N // TI + LAG + 2,),
        in_specs=[
            pl.BlockSpec((TI, D), clamp2),
            pl.BlockSpec((K, D), lambda i: (0, 0)),
            pl.BlockSpec((1, 1, TI), clamp3),
            pl.BlockSpec((K, 1), lambda i: (0, 0)),
        ],
        out_specs=[
            pl.BlockSpec((1, 1, TI), clamp3),
            pl.BlockSpec(memory_space=pl.ANY),
            pl.BlockSpec((K, 1), lambda i: (0, 0)),
        ],
        out_shape=[
            jax.ShapeDtypeStruct((N // TI, 1, TI), jnp.int32),
            jax.ShapeDtypeStruct((B, K, HW), jnp.float32),
            jax.ShapeDtypeStruct((K, 1), jnp.float32),
        ],
        scratch_shapes=[
            pltpu.VMEM((N // TI, 1, TI), jnp.int32),
            pltpu.VMEM((K, 1), jnp.float32),
            pltpu.VMEM((2, KB, HW), jnp.float32),
            pltpu.SemaphoreType.DMA((2,)),
        ],
    )(flat, codebook, x2, y2)


def qst_body(x_ref, q_ref, cnt_ref, o_ref, loss_ref, perp_ref, acc_ref):
    b = pl.program_id(0)
    x = x_ref[0]
    q = q_ref[0]
    qst = x + (q - x)
    o_ref[0] = qst
    diff = qst - x
    part = jnp.sum(diff * diff)

    @pl.when(b == 0)
    def _():
        acc_ref[0, 0] = part

    @pl.when(b != 0)
    def _():
        acc_ref[0, 0] = acc_ref[0, 0] + part

    @pl.when(b == pl.num_programs(0) - 1)
    def _():
        m = acc_ref[0, 0] / jnp.float32(N * D)
        loss_ref[0, 0] = m + 0.25 * m
        p = cnt_ref[...] * jnp.float32(1.0 / N)
        t = p * jnp.log(p + 1e-10)
        perp_ref[0, 0] = jnp.exp(-jnp.sum(t))


def _qst_call(flat3, q3, cnt):
    return pl.pallas_call(
        qst_body,
        grid=(B,),
        in_specs=[
            pl.BlockSpec((1, HW, D), lambda b: (b, 0, 0)),
            pl.BlockSpec((1, HW, D), lambda b: (b, 0, 0)),
            pl.BlockSpec((K // 128, 128), lambda b: (0, 0)),
        ],
        out_specs=[
            pl.BlockSpec((1, HW, D), lambda b: (b, 0, 0)),
            pl.BlockSpec(memory_space=pltpu.SMEM),
            pl.BlockSpec(memory_space=pltpu.SMEM),
        ],
        out_shape=[
            jax.ShapeDtypeStruct((B, HW, D), jnp.float32),
            jax.ShapeDtypeStruct((1, 1), jnp.float32),
            jax.ShapeDtypeStruct((1, 1), jnp.float32),
        ],
        scratch_shapes=[pltpu.SMEM((1, 1), jnp.float32)],
    )(flat3, q3, cnt)


def _sc_gather(codebook, idx):
    info = plsc.get_sparse_core_info()
    nc, ns = info.num_cores, info.num_subcores
    nw = nc * ns
    bpw = N // nw
    mesh = plsc.VectorSubcoreMesh(core_axis_name="c", subcore_axis_name="s")

    @functools.partial(
        pl.kernel,
        mesh=mesh,
        out_type=jax.ShapeDtypeStruct((N, D), jnp.float32),
        scratch_types=[
            pltpu.VMEM((bpw,), jnp.int32),
            pltpu.VMEM((bpw, D), jnp.float32),
            pltpu.SemaphoreType.DMA,
        ],
    )
    def gather_k(table_hbm, idx_hbm, out_hbm, idx_v, rows_v, sem):
        wid = lax.axis_index("s") * nc + lax.axis_index("c")
        base = wid * bpw
        pltpu.sync_copy(idx_hbm.at[pl.ds(base, bpw)], idx_v)
        pltpu.async_copy(table_hbm.at[idx_v], rows_v, sem).wait()
        pltpu.sync_copy(rows_v, out_hbm.at[pl.ds(base, bpw)])

    return gather_k(codebook, idx)


def kernel(inputs, codebook):
    flat = jnp.transpose(inputs, (0, 2, 3, 1)).reshape(-1, D)
    x2 = jnp.sum(flat * flat, axis=1, keepdims=True).reshape(N // TI, 1, TI)
    y2 = jnp.sum(codebook * codebook, axis=1).reshape(K, 1)

    idx, enc, cnt = _fused_call(flat, codebook, x2, y2)

    qflat = _sc_gather(codebook, idx.reshape(N))
    qst3, loss, perp = _qst_call(flat.reshape(B, HW, D), qflat.reshape(B, HW, D),
                                 cnt.reshape(K // 128, 128))

    quantized_st = jnp.transpose(qst3.reshape(B, 32, 32, D), (0, 3, 1, 2))
    encodings_out = enc.reshape(B, K, 32, 32)
    return (quantized_st, loss.reshape(()), perp.reshape(()), encodings_out)
